# baseline (speedup 1.0000x reference)
import functools

import jax
import jax.numpy as jnp
from jax import lax
from jax.experimental import pallas as pl
from jax.experimental.pallas import tpu as pltpu
from jax.experimental.pallas import tpu_sc as plsc

N_REQ = 4096
ROW = 32768
NC = 2
NS = 16
NW = NC * NS
B_PER_W = N_REQ // NW
L = 16


def _sc_body(table_hbm, pool_hbm, prefix_hbm, out_hbm,
             pool_v, prefix_v, idx_v, gath_v, sem):
    wid = lax.axis_index("s") * NC + lax.axis_index("c")
    base = wid * B_PER_W
    pltpu.sync_copy(pool_hbm.at[pl.ds(base, B_PER_W)], pool_v)
    pltpu.sync_copy(prefix_hbm.at[pl.ds(base, B_PER_W)], prefix_v)
    for i in range(B_PER_W // L):
        s = pl.ds(i * L, L)
        p = pool_v[s]
        f = prefix_v[s]
        idx_v[s] = p * ROW + jnp.maximum(f - 1, 0)
    pltpu.async_copy(table_hbm.at[idx_v], gath_v, sem).wait()
    for i in range(B_PER_W // L):
        s = pl.ds(i * L, L)
        gath_v[s] = jnp.where(prefix_v[s] > 0, gath_v[s], jnp.int32(-1))
    pltpu.sync_copy(gath_v, out_hbm.at[pl.ds(base, B_PER_W)])


@jax.jit
def _last_loc(table_flat, pool_idx, prefix_lens):
    mesh = plsc.VectorSubcoreMesh(core_axis_name="c", subcore_axis_name="s")
    return pl.kernel(
        _sc_body,
        mesh=mesh,
        out_type=jax.ShapeDtypeStruct((N_REQ,), jnp.int32),
        scratch_types=[
            pltpu.VMEM((B_PER_W,), jnp.int32),
            pltpu.VMEM((B_PER_W,), jnp.int32),
            pltpu.VMEM((B_PER_W,), jnp.int32),
            pltpu.VMEM((B_PER_W,), jnp.int32),
            pltpu.SemaphoreType.DMA,
        ],
    )(table_flat, pool_idx, prefix_lens)


def kernel(req_to_token, req_pool_indices_tensor, prefix_lens_tensor):
    table_flat = req_to_token.reshape(-1)
    return _last_loc(table_flat, req_pool_indices_tensor, prefix_lens_tensor)


# speedup vs baseline: 17.9677x; 17.9677x over previous
import jax
import jax.numpy as jnp
from jax import lax
from jax.experimental import pallas as pl
from jax.experimental.pallas import tpu as pltpu
from jax.experimental.pallas import tpu_sc as plsc

N_REQ = 4096
ROW = 32768
SUB = 8
LANE = 128
N_VROWS = N_REQ * ROW // LANE
NC = 2
NS = 16
NW = NC * NS
B_PER_W = N_REQ // NW
L = 16


def _sc_body(table_hbm, pool_hbm, prefix_hbm, out_hbm,
             pool_v, prefix_v, lane_v, idx_v, rows_v, out_v, sem):
    wid = lax.axis_index("s") * NC + lax.axis_index("c")
    base = wid * B_PER_W
    pltpu.sync_copy(pool_hbm.at[pl.ds(base, B_PER_W)], pool_v)
    pltpu.sync_copy(prefix_hbm.at[pl.ds(base, B_PER_W)], prefix_v)
    for i in range(B_PER_W // L):
        s = pl.ds(i * L, L)
        p = pool_v[s]
        f = prefix_v[s]
        col = jnp.maximum(f - 1, 0)
        idx_v[s] = ((p >> 3) * (ROW // LANE) + (col >> 7)) * SUB + (p & (SUB - 1))
        lane_v[s] = col & (LANE - 1)
    pltpu.async_copy(table_hbm.at[idx_v], rows_v, sem).wait()
    for i in range(B_PER_W // L):
        s = pl.ds(i * L, L)
        rid = lax.iota(jnp.int32, L) + jnp.int32(i * L)
        g = plsc.load_gather(rows_v, [rid, lane_v[s]])
        out_v[s] = jnp.where(prefix_v[s] > 0, g, jnp.int32(-1))
    pltpu.sync_copy(out_v, out_hbm.at[pl.ds(base, B_PER_W)])


@jax.jit
def _last_loc(table_rows, pool_idx, prefix_lens):
    mesh = plsc.VectorSubcoreMesh(core_axis_name="c", subcore_axis_name="s")
    return pl.kernel(
        _sc_body,
        mesh=mesh,
        out_type=jax.ShapeDtypeStruct((N_REQ,), jnp.int32),
        scratch_types=[
            pltpu.VMEM((B_PER_W,), jnp.int32),
            pltpu.VMEM((B_PER_W,), jnp.int32),
            pltpu.VMEM((B_PER_W,), jnp.int32),
            pltpu.VMEM((B_PER_W,), jnp.int32),
            pltpu.VMEM((B_PER_W, LANE), jnp.int32),
            pltpu.VMEM((B_PER_W,), jnp.int32),
            pltpu.SemaphoreType.DMA,
        ],
        compiler_params=pltpu.CompilerParams(needs_layout_passes=False),
    )(table_rows, pool_idx, prefix_lens)


def kernel(req_to_token, req_pool_indices_tensor, prefix_lens_tensor):
    r, c = req_to_token.shape
    table_rows = (
        req_to_token.reshape(r // SUB, SUB, c // LANE, LANE)
        .transpose(0, 2, 1, 3)
        .reshape(r * c // LANE, LANE)
    )
    return _last_loc(table_rows, req_pool_indices_tensor, prefix_lens_tensor)


# speedup vs baseline: 18.7942x; 1.0460x over previous
import jax
import jax.numpy as jnp
from jax import lax
from jax.experimental import pallas as pl
from jax.experimental.pallas import tpu as pltpu
from jax.experimental.pallas import tpu_sc as plsc

N_REQ = 4096
ROW = 32768
SUB = 8
LANE = 128
NC = 2
NS = 16
NW = NC * NS
B_PER_W = N_REQ // NW
L = 16


def _sc_body(table_hbm, pool_hbm, prefix_hbm, out_hbm,
             pool_v, prefix_v, idx_v, gath_v, sem):
    wid = lax.axis_index("s") * NC + lax.axis_index("c")
    base = wid * B_PER_W
    pltpu.sync_copy(pool_hbm.at[pl.ds(base, B_PER_W)], pool_v)
    pltpu.sync_copy(prefix_hbm.at[pl.ds(base, B_PER_W)], prefix_v)
    for i in range(B_PER_W // L):
        s = pl.ds(i * L, L)
        p = pool_v[s]
        col = jnp.maximum(prefix_v[s] - 1, 0)
        idx_v[s] = (((p >> 3) * (ROW // LANE) + (col >> 7)) * (SUB * LANE)
                    + ((p & (SUB - 1)) << 7) + (col & (LANE - 1)))
    pltpu.async_copy(table_hbm.at[idx_v], gath_v, sem).wait()
    for i in range(B_PER_W // L):
        s = pl.ds(i * L, L)
        gath_v[s] = jnp.where(prefix_v[s] > 0, gath_v[s], jnp.int32(-1))
    pltpu.sync_copy(gath_v, out_hbm.at[pl.ds(base, B_PER_W)])


@jax.jit
def _last_loc(table_flat, pool_idx, prefix_lens):
    mesh = plsc.VectorSubcoreMesh(core_axis_name="c", subcore_axis_name="s")
    return pl.kernel(
        _sc_body,
        mesh=mesh,
        out_type=jax.ShapeDtypeStruct((N_REQ,), jnp.int32),
        scratch_types=[
            pltpu.VMEM((B_PER_W,), jnp.int32),
            pltpu.VMEM((B_PER_W,), jnp.int32),
            pltpu.VMEM((B_PER_W,), jnp.int32),
            pltpu.VMEM((B_PER_W,), jnp.int32),
            pltpu.SemaphoreType.DMA,
        ],
        compiler_params=pltpu.CompilerParams(needs_layout_passes=False),
    )(table_flat, pool_idx, prefix_lens)


def kernel(req_to_token, req_pool_indices_tensor, prefix_lens_tensor):
    r, c = req_to_token.shape
    table_flat = (
        req_to_token.reshape(r // SUB, SUB, c // LANE, LANE)
        .transpose(0, 2, 1, 3)
        .reshape(r * c)
    )
    return _last_loc(table_flat, req_pool_indices_tensor, prefix_lens_tensor)


# speedup vs baseline: 19.2490x; 1.0242x over previous
import jax
import jax.numpy as jnp
from jax import lax
from jax.experimental import pallas as pl
from jax.experimental.pallas import tpu as pltpu
from jax.experimental.pallas import tpu_sc as plsc

N_REQ = 4096
ROW = 32768
SUB = 8
LANE = 128
NC = 2
NS = 16
NW = NC * NS
B_PER_W = N_REQ // NW
L = 16


def _sc_body(table_hbm, pool_hbm, prefix_hbm, out_hbm,
             pool_v, prefix_v, idx_v, gath_v, sem, sem2):
    wid = lax.axis_index("s") * NC + lax.axis_index("c")
    base = wid * B_PER_W
    cp_pool = pltpu.async_copy(pool_hbm.at[pl.ds(base, B_PER_W)], pool_v, sem)
    cp_pref = pltpu.async_copy(prefix_hbm.at[pl.ds(base, B_PER_W)], prefix_v, sem2)
    cp_pool.wait()
    cp_pref.wait()
    for i in range(B_PER_W // L):
        s = pl.ds(i * L, L)
        p = pool_v[s]
        col = jnp.maximum(prefix_v[s] - 1, 0)
        idx_v[s] = (((p >> 3) * (ROW // LANE) + (col >> 7)) * (SUB * LANE)
                    + ((p & (SUB - 1)) << 7) + (col & (LANE - 1)))
    pltpu.async_copy(table_hbm.at[idx_v], gath_v, sem).wait()
    for i in range(B_PER_W // L):
        s = pl.ds(i * L, L)
        gath_v[s] = jnp.where(prefix_v[s] > 0, gath_v[s], jnp.int32(-1))
    pltpu.sync_copy(gath_v, out_hbm.at[pl.ds(base, B_PER_W)])


@jax.jit
def _last_loc(table_flat, pool_idx, prefix_lens):
    mesh = plsc.VectorSubcoreMesh(core_axis_name="c", subcore_axis_name="s")
    return pl.kernel(
        _sc_body,
        mesh=mesh,
        out_type=jax.ShapeDtypeStruct((N_REQ,), jnp.int32),
        scratch_types=[
            pltpu.VMEM((B_PER_W,), jnp.int32),
            pltpu.VMEM((B_PER_W,), jnp.int32),
            pltpu.VMEM((B_PER_W,), jnp.int32),
            pltpu.VMEM((B_PER_W,), jnp.int32),
            pltpu.SemaphoreType.DMA,
            pltpu.SemaphoreType.DMA,
        ],
        compiler_params=pltpu.CompilerParams(
            needs_layout_passes=False, skip_device_barrier=True),
    )(table_flat, pool_idx, prefix_lens)


def kernel(req_to_token, req_pool_indices_tensor, prefix_lens_tensor):
    r, c = req_to_token.shape
    table_flat = (
        req_to_token.reshape(r // SUB, SUB, c // LANE, LANE)
        .transpose(0, 2, 1, 3)
        .reshape(r * c)
    )
    return _last_loc(table_flat, req_pool_indices_tensor, prefix_lens_tensor)
